# Initial kernel scaffold; baseline (speedup 1.0000x reference)
import jax
import jax.numpy as jnp
from jax.experimental import pallas as pl


def kernel(x, adj, in_batch, cluster, W_g1, b_g1, W_g2, b_g2, W_l1, b_l1, W_l2, b_l2, W_m1, b_m1, W_o1, b_o1, W_fc):
    raise NotImplementedError("write your pallas kernel here")


# speedup vs baseline: 49.1680x; 49.1680x over previous
import functools
import jax
import jax.numpy as jnp
from jax import lax
from jax.experimental import pallas as pl
from jax.experimental.pallas import tpu as pltpu
from jax.experimental.pallas import tpu_sc as plsc

N = 10000
E = 320000
C = 256
NW = 32
EPW = E // NW
NCHUNK = 80
KE = EPW // NCHUNK
ACCN = 10240
RPS = ACCN // 16
PR = 272

_mesh = plsc.VectorSubcoreMesh(core_axis_name="c", subcore_axis_name="s")


def _wid():
    return lax.axis_index("s") * 2 + lax.axis_index("c")


@functools.partial(
    pl.kernel,
    mesh=_mesh,
    compiler_params=pltpu.CompilerParams(needs_layout_passes=False, use_tc_tiling_on_sc=False),
    out_type=[
        jax.ShapeDtypeStruct((NW, N), jnp.float32),
        jax.ShapeDtypeStruct((NW, C * C), jnp.float32),
    ],
    scratch_types=[
        pltpu.VMEM((N,), jnp.int32),
        pltpu.VMEM((EPW,), jnp.int32),
        pltpu.VMEM((EPW,), jnp.int32),
        pltpu.VMEM((N,), jnp.float32),
        pltpu.VMEM((C * C,), jnp.float32),
    ],
)
def _sc_precompute(src_h, dst_h, clus_h, zer_h, degp_h, bmp_h,
                   clus_v, src_v, dst_v, deg_v, bm_v):
    w = _wid()
    pltpu.sync_copy(clus_h, clus_v)
    pltpu.sync_copy(src_h.at[w], src_v)
    pltpu.sync_copy(dst_h.at[w], dst_v)
    pltpu.sync_copy(zer_h, bm_v)
    pltpu.sync_copy(zer_h.at[pl.ds(0, N)], deg_v)
    ones = jnp.full((16,), 1.0, jnp.float32)

    def body(j, carry):
        s16 = src_v[pl.ds(j * 16, 16)]
        d16 = dst_v[pl.ds(j * 16, 16)]
        plsc.addupdate_scatter(deg_v, [d16], ones)
        cs = plsc.load_gather(clus_v, [s16])
        cd = plsc.load_gather(clus_v, [d16])
        key = cs * C + cd
        plsc.addupdate_scatter(bm_v, [key], ones)
        return carry

    lax.fori_loop(0, EPW // 16, body, 0)
    pltpu.sync_copy(deg_v, degp_h.at[w])
    pltpu.sync_copy(bm_v, bmp_h.at[w])


@functools.partial(
    pl.kernel,
    mesh=_mesh,
    compiler_params=pltpu.CompilerParams(needs_layout_passes=False, use_tc_tiling_on_sc=False),
    out_type=jax.ShapeDtypeStruct((2, ACCN, 64), jnp.float32),
    scratch_types=[
        pltpu.VMEM((NCHUNK, KE), jnp.int32),
        pltpu.VMEM((NCHUNK, KE), jnp.int32),
        pltpu.VMEM((KE, 64), jnp.float32),
        pltpu.VMEM_SHARED((ACCN, 64), jnp.float32),
        pltpu.SemaphoreType.DMA,
    ],
)
def _sc_spmm(xws_h, src_h, dst_h, zrow_h, out_h,
             src_v, dst_v, rows_v, acc_sh, sem):
    cid = lax.axis_index("c")
    sid = lax.axis_index("s")
    w = _wid()
    pltpu.sync_copy(src_h.at[w], src_v)
    pltpu.sync_copy(dst_h.at[w], dst_v)
    pltpu.sync_copy(zrow_h, acc_sh.at[pl.ds(sid * RPS, RPS)])
    plsc.subcore_barrier()

    def body(j, carry):
        pltpu.async_copy(xws_h.at[src_v.at[j]], rows_v, sem).wait()
        pltpu.sync_copy(rows_v, acc_sh.at[dst_v.at[j]], add=True)
        return carry

    lax.fori_loop(0, NCHUNK, body, 0)
    plsc.subcore_barrier()
    pltpu.sync_copy(acc_sh.at[pl.ds(sid * RPS, RPS)],
                    out_h.at[cid, pl.ds(sid * RPS, RPS)])


@functools.partial(
    pl.kernel,
    mesh=_mesh,
    compiler_params=pltpu.CompilerParams(needs_layout_passes=False, use_tc_tiling_on_sc=False),
    out_type=[
        jax.ShapeDtypeStruct((PR, 256), jnp.float32),
        jax.ShapeDtypeStruct((PR, 16), jnp.float32),
    ],
    scratch_types=[
        pltpu.VMEM((5, 125), jnp.int32),
        pltpu.VMEM((125, 256), jnp.float32),
        pltpu.VMEM((125, 16), jnp.float32),
        pltpu.VMEM_SHARED((PR, 256), jnp.float32),
        pltpu.VMEM_SHARED((PR, 16), jnp.float32),
    ],
)
def _sc_pool(y_h, cidx_h, zrow_h, one_h, z16_h, pool_h, cnt_h,
             cidx_v, rows_v, ones_v, acc_sh, cacc_sh):
    cid = lax.axis_index("c")
    sid = lax.axis_index("s")

    @pl.when(cid == 0)
    def _():
        pltpu.sync_copy(cidx_h.at[sid], cidx_v)
        pltpu.sync_copy(one_h, ones_v)
        pltpu.sync_copy(zrow_h, acc_sh.at[pl.ds(sid * 17, 17)])
        pltpu.sync_copy(z16_h, cacc_sh.at[pl.ds(sid * 17, 17)])

    plsc.subcore_barrier()

    @pl.when(cid == 0)
    def _():
        def body(j, carry):
            pltpu.sync_copy(y_h.at[pl.ds(sid * 625 + j * 125, 125)], rows_v)
            pltpu.sync_copy(rows_v, acc_sh.at[cidx_v.at[j]], add=True)
            pltpu.sync_copy(ones_v, cacc_sh.at[cidx_v.at[j]], add=True)
            return carry

        lax.fori_loop(0, 5, body, 0)

    plsc.subcore_barrier()

    @pl.when(cid == 0)
    def _():
        pltpu.sync_copy(acc_sh.at[pl.ds(sid * 17, 17)],
                        pool_h.at[pl.ds(sid * 17, 17)])
        pltpu.sync_copy(cacc_sh.at[pl.ds(sid * 17, 17)],
                        cnt_h.at[pl.ds(sid * 17, 17)])


def _dot_bf16(a, b):
    return jnp.dot(a.astype(jnp.bfloat16), b.astype(jnp.bfloat16),
                   preferred_element_type=jnp.float32)


def _elu(x):
    xn = jnp.minimum(x, 0.0)
    poly = xn * (1.0 + xn * (0.5 + xn * (1.0 / 6.0 + xn * (1.0 / 24.0
                 + xn * (1.0 / 120.0)))))
    em1 = jnp.where(xn > -0.125, poly, jnp.exp(xn) - 1.0)
    return jnp.where(x > 0, x, em1)


def _t1_body(x_ref, w_ref, degp_ref, xws_ref, o_dinv_ref):
    deg = jnp.sum(degp_ref[0], axis=0) + 1.0
    dv = 1.0 / jnp.sqrt(deg)
    xw = jnp.dot(x_ref[...].astype(jnp.bfloat16),
                 w_ref[...].astype(jnp.bfloat16),
                 preferred_element_type=jnp.float32)
    xws_ref[...] = dv[:, None] * xw
    o_dinv_ref[0] = jnp.broadcast_to(dv[None, :], (8, dv.shape[0]))


def _t2_body(acc_ref, xws_ref, b_ref, dinv_ref, hs_ref):
    dv = dinv_ref[0, 0, :]
    s = acc_ref[0] + acc_ref[1] + xws_ref[...]
    h = _elu(dv[:, None] * s + b_ref[...])
    hb = h.astype(jnp.bfloat16).astype(jnp.float32)
    hs_ref[...] = dv[:, None] * hb


def _t3_body(acc_ref, hs_ref, w_ref, b_ref, dinv_ref, h2_ref, s1_ref, s2_ref):
    i = pl.program_id(0)
    dv = dinv_ref[0, 0, :]
    ah = dv[:, None] * (acc_ref[0] + acc_ref[1] + hs_ref[...])
    wb = w_ref[...].astype(jnp.bfloat16).astype(jnp.float32)
    h2 = _elu(jnp.dot(ah, wb, precision=lax.Precision.HIGHEST,
                      preferred_element_type=jnp.float32)
              + b_ref[...])
    h2_ref[...] = h2

    @pl.when(i == 0)
    def _():
        s1_ref[...] = jnp.zeros_like(s1_ref)
        s2_ref[...] = jnp.zeros_like(s2_ref)

    s1_ref[...] += jnp.broadcast_to(jnp.sum(h2, axis=0)[None, :], s1_ref.shape)
    s2_ref[...] += jnp.broadcast_to(jnp.sum(h2 * h2, axis=0)[None, :],
                                    s2_ref.shape)


def _t4a_body(h2_ref, s1_ref, sq_ref):
    i = pl.program_id(0)
    mean = s1_ref[0, :] * (1.0 / N)
    d = h2_ref[...] - mean[None, :]

    @pl.when(i == 0)
    def _():
        sq_ref[...] = jnp.zeros_like(sq_ref)

    sq_ref[...] += jnp.broadcast_to(jnp.sum(d * d, axis=0)[None, :],
                                    sq_ref.shape)


def _t4_body(h2_ref, s1_ref, sq_ref, y_ref):
    mean = s1_ref[0, :] * (1.0 / N)
    var = sq_ref[0, :] * (1.0 / N)
    y_ref[...] = (h2_ref[...] - mean[None, :]) / jnp.sqrt(var + 1e-5)[None, :]


def _t5_body(bmp_ref, pool_ref, cnt_ref, wl1_ref, bl1_ref, wl2_ref, bl2_ref,
             wm1_ref, bm1_ref, wo1_ref, bo1_ref, g_ref):
    bm = jnp.sum(bmp_ref[...], axis=0)
    at = bm.T
    r = lax.broadcasted_iota(jnp.int32, (C, C), 0)
    c = lax.broadcasted_iota(jnp.int32, (C, C), 1)
    a = jnp.where(r == c, 1.0, jnp.where(at > 0, 1.0, 0.0))
    dc = 1.0 / jnp.sqrt(jnp.sum(a, axis=1))
    m = dc[:, None] * a * dc[None, :]
    px = pool_ref[0:C, :] / jnp.maximum(cnt_ref[0:C, 0:1], 1.0)
    z = _elu(jnp.dot(m, _dot_bf16(px, wl1_ref[...]),
                     precision=lax.Precision.HIGHEST, preferred_element_type=jnp.float32) + bl1_ref[...])
    z = _elu(jnp.dot(m, _dot_bf16(z, wl2_ref[...]),
                     precision=lax.Precision.HIGHEST, preferred_element_type=jnp.float32) + bl2_ref[...])
    z = _elu(jnp.dot(m, _dot_bf16(z, wm1_ref[...]),
                     precision=lax.Precision.HIGHEST, preferred_element_type=jnp.float32) + bm1_ref[...])
    g_ref[...] = jnp.dot(m, _dot_bf16(z, wo1_ref[...]),
                         precision=lax.Precision.HIGHEST, preferred_element_type=jnp.float32) + bo1_ref[...]


def _t6_body(w_ref, g_ref, o_ref):
    o_ref[...] = _dot_bf16(w_ref[...], g_ref[...])


def kernel(x, adj, in_batch, cluster, W_g1, b_g1, W_g2, b_g2, W_l1, b_l1,
           W_l2, b_l2, W_m1, b_m1, W_o1, b_o1, W_fc):
    f32 = jnp.float32
    src = adj[0]
    dst = adj[1]
    src2 = src.reshape(NW, EPW)
    dst2 = dst.reshape(NW, EPW)
    src3 = src.reshape(NW, NCHUNK, KE)
    dst3 = dst.reshape(NW, NCHUNK, KE)
    cidx3 = cluster.reshape(16, 5, 125)
    zbm = jnp.zeros((C * C,), f32)
    zrow = jnp.zeros((RPS, 64), f32)
    zrow256 = jnp.zeros((17, 256), f32)
    z16 = jnp.zeros((17, 16), f32)
    ones125 = jnp.ones((125, 16), f32)

    degp, bmp = _sc_precompute(src2, dst2, cluster, zbm)

    BN = 1000
    GN = N // BN
    degT = degp.reshape(NW, GN, BN).transpose(1, 0, 2)
    t1 = pl.pallas_call(
        _t1_body,
        grid=(GN,),
        in_specs=[
            pl.BlockSpec((BN, 128), lambda i: (i, 0)),
            pl.BlockSpec((128, 64), lambda i: (0, 0)),
            pl.BlockSpec((1, NW, BN), lambda i: (i, 0, 0)),
        ],
        out_specs=[
            pl.BlockSpec((BN, 64), lambda i: (i, 0)),
            pl.BlockSpec((1, 8, BN), lambda i: (i, 0, 0)),
        ],
        out_shape=[
            jax.ShapeDtypeStruct((N, 64), f32),
            jax.ShapeDtypeStruct((GN, 8, BN), f32),
        ],
    )
    xws1, dinv8 = t1(x, W_g1, degT)

    accp1 = _sc_spmm(xws1, src3, dst3, zrow)

    t2 = pl.pallas_call(
        _t2_body,
        grid=(GN,),
        in_specs=[
            pl.BlockSpec((2, BN, 64), lambda i: (0, i, 0)),
            pl.BlockSpec((BN, 64), lambda i: (i, 0)),
            pl.BlockSpec((1, 64), lambda i: (0, 0)),
            pl.BlockSpec((1, 8, BN), lambda i: (i, 0, 0)),
        ],
        out_specs=pl.BlockSpec((BN, 64), lambda i: (i, 0)),
        out_shape=jax.ShapeDtypeStruct((N, 64), f32),
    )
    hs1 = t2(accp1, xws1, b_g1.reshape(1, 64), dinv8)

    accp2 = _sc_spmm(hs1, src3, dst3, zrow)

    t3 = pl.pallas_call(
        _t3_body,
        grid=(GN,),
        in_specs=[
            pl.BlockSpec((2, BN, 64), lambda i: (0, i, 0)),
            pl.BlockSpec((BN, 64), lambda i: (i, 0)),
            pl.BlockSpec((64, 256), lambda i: (0, 0)),
            pl.BlockSpec((1, 256), lambda i: (0, 0)),
            pl.BlockSpec((1, 8, BN), lambda i: (i, 0, 0)),
        ],
        out_specs=[
            pl.BlockSpec((BN, 256), lambda i: (i, 0)),
            pl.BlockSpec((8, 256), lambda i: (0, 0)),
            pl.BlockSpec((8, 256), lambda i: (0, 0)),
        ],
        out_shape=[
            jax.ShapeDtypeStruct((N, 256), f32),
            jax.ShapeDtypeStruct((8, 256), f32),
            jax.ShapeDtypeStruct((8, 256), f32),
        ],
    )
    h2, s1, s2 = t3(accp2, hs1, W_g2, b_g2.reshape(1, 256), dinv8)

    t4a = pl.pallas_call(
        _t4a_body,
        grid=(GN,),
        in_specs=[
            pl.BlockSpec((BN, 256), lambda i: (i, 0)),
            pl.BlockSpec((8, 256), lambda i: (0, 0)),
        ],
        out_specs=pl.BlockSpec((8, 256), lambda i: (0, 0)),
        out_shape=jax.ShapeDtypeStruct((8, 256), f32),
    )
    sq = t4a(h2, s1)

    t4 = pl.pallas_call(
        _t4_body,
        grid=(GN,),
        in_specs=[
            pl.BlockSpec((BN, 256), lambda i: (i, 0)),
            pl.BlockSpec((8, 256), lambda i: (0, 0)),
            pl.BlockSpec((8, 256), lambda i: (0, 0)),
        ],
        out_specs=pl.BlockSpec((BN, 256), lambda i: (i, 0)),
        out_shape=jax.ShapeDtypeStruct((N, 256), f32),
    )
    y = t4(h2, s1, sq)

    pool, cntl = _sc_pool(y, cidx3, zrow256, ones125, z16)

    t5 = pl.pallas_call(
        _t5_body,
        in_specs=[
            pl.BlockSpec((NW, C, C), lambda: (0, 0, 0)),
            pl.BlockSpec((PR, 256), lambda: (0, 0)),
            pl.BlockSpec((PR, 16), lambda: (0, 0)),
            pl.BlockSpec((256, 128), lambda: (0, 0)),
            pl.BlockSpec((1, 128), lambda: (0, 0)),
            pl.BlockSpec((128, 64), lambda: (0, 0)),
            pl.BlockSpec((1, 64), lambda: (0, 0)),
            pl.BlockSpec((64, 16), lambda: (0, 0)),
            pl.BlockSpec((1, 16), lambda: (0, 0)),
            pl.BlockSpec((16, 8), lambda: (0, 0)),
            pl.BlockSpec((1, 8), lambda: (0, 0)),
        ],
        out_specs=pl.BlockSpec((C, 8), lambda: (0, 0)),
        out_shape=jax.ShapeDtypeStruct((C, 8), f32),
    )
    wo1p = jnp.pad(W_o1, ((0, 0), (0, 3)))
    bo1p = jnp.pad(b_o1, (0, 3)).reshape(1, 8)
    gpad = t5(bmp.reshape(NW, C, C), pool, cntl, W_l1, b_l1.reshape(1, 128),
              W_l2, b_l2.reshape(1, 64), W_m1, b_m1.reshape(1, 16),
              wo1p, bo1p)
    g = gpad[:, :5].reshape(-1)

    BM = 1000
    t6 = pl.pallas_call(
        _t6_body,
        grid=(30000 // BM,),
        in_specs=[
            pl.BlockSpec((BM, 1280), lambda i: (i, 0)),
            pl.BlockSpec((1280, 1), lambda i: (0, 0)),
        ],
        out_specs=pl.BlockSpec((BM, 1), lambda i: (i, 0)),
        out_shape=jax.ShapeDtypeStruct((30000, 1), f32),
    )
    o = t6(W_fc, g.reshape(1280, 1)).reshape(-1)
    return (g, o)


# speedup vs baseline: 60.7825x; 1.2362x over previous
import functools
import jax
import jax.numpy as jnp
from jax import lax
from jax.experimental import pallas as pl
from jax.experimental.pallas import tpu as pltpu
from jax.experimental.pallas import tpu_sc as plsc

N = 10000
E = 320000
C = 256
NW = 32
EPW = E // NW
NCHUNK = 80
KE = EPW // NCHUNK
ACCN = 10240
RPS = ACCN // 16
PR = 272

_mesh = plsc.VectorSubcoreMesh(core_axis_name="c", subcore_axis_name="s")


def _wid():
    return lax.axis_index("s") * 2 + lax.axis_index("c")


@functools.partial(
    pl.kernel,
    mesh=_mesh,
    compiler_params=pltpu.CompilerParams(needs_layout_passes=False, use_tc_tiling_on_sc=False),
    out_type=[
        jax.ShapeDtypeStruct((10, NW, 1000), jnp.float32),
        jax.ShapeDtypeStruct((NW, C * C), jnp.float32),
    ],
    scratch_types=[
        pltpu.VMEM((N,), jnp.int32),
        pltpu.VMEM((EPW,), jnp.int32),
        pltpu.VMEM((EPW,), jnp.int32),
        pltpu.VMEM((N,), jnp.float32),
        pltpu.VMEM((C * C,), jnp.float32),
    ],
)
def _sc_precompute(src_h, dst_h, clus_h, zer_h, degp_h, bmp_h,
                   clus_v, src_v, dst_v, deg_v, bm_v):
    w = _wid()
    pltpu.sync_copy(clus_h, clus_v)
    pltpu.sync_copy(src_h.at[w], src_v)
    pltpu.sync_copy(dst_h.at[w], dst_v)
    pltpu.sync_copy(zer_h, bm_v)
    pltpu.sync_copy(zer_h.at[pl.ds(0, N)], deg_v)
    ones = jnp.full((16,), 1.0, jnp.float32)

    def body(j, carry):
        s16 = src_v[pl.ds(j * 16, 16)]
        d16 = dst_v[pl.ds(j * 16, 16)]
        plsc.addupdate_scatter(deg_v, [d16], ones)
        cs = plsc.load_gather(clus_v, [s16])
        cd = plsc.load_gather(clus_v, [d16])
        key = cs * C + cd
        plsc.addupdate_scatter(bm_v, [key], ones)
        return carry

    lax.fori_loop(0, EPW // 16, body, 0)
    for i in range(10):
        pltpu.sync_copy(deg_v.at[pl.ds(i * 1000, 1000)], degp_h.at[i, w])
    pltpu.sync_copy(bm_v, bmp_h.at[w])


@functools.partial(
    pl.kernel,
    mesh=_mesh,
    compiler_params=pltpu.CompilerParams(needs_layout_passes=False, use_tc_tiling_on_sc=False),
    out_type=jax.ShapeDtypeStruct((2, ACCN, 64), jnp.float32),
    scratch_types=[
        pltpu.VMEM((NCHUNK, KE), jnp.int32),
        pltpu.VMEM((NCHUNK, KE), jnp.int32),
        pltpu.VMEM((KE, 64), jnp.float32),
        pltpu.VMEM((KE, 64), jnp.float32),
        pltpu.VMEM_SHARED((ACCN, 64), jnp.float32),
        pltpu.SemaphoreType.DMA,
        pltpu.SemaphoreType.DMA,
    ],
)
def _sc_spmm(xws_h, src_h, dst_h, zrow_h, out_h,
             src_v, dst_v, rows0_v, rows1_v, acc_sh, sem0, sem1):
    cid = lax.axis_index("c")
    sid = lax.axis_index("s")
    w = _wid()
    pltpu.sync_copy(src_h.at[w], src_v)
    pltpu.sync_copy(dst_h.at[w], dst_v)
    pltpu.sync_copy(zrow_h, acc_sh.at[pl.ds(sid * RPS, RPS)])
    plsc.subcore_barrier()

    pltpu.async_copy(xws_h.at[src_v.at[0]], rows0_v, sem0)

    def body(i, carry):
        j0 = 2 * i
        pltpu.async_copy(xws_h.at[src_v.at[j0 + 1]], rows1_v, sem1)
        pltpu.make_async_copy(xws_h.at[src_v.at[j0]], rows0_v, sem0).wait()
        pltpu.sync_copy(rows0_v, acc_sh.at[dst_v.at[j0]], add=True)

        @pl.when(i < NCHUNK // 2 - 1)
        def _():
            pltpu.async_copy(xws_h.at[src_v.at[j0 + 2]], rows0_v, sem0)

        pltpu.make_async_copy(xws_h.at[src_v.at[j0 + 1]], rows1_v, sem1).wait()
        pltpu.sync_copy(rows1_v, acc_sh.at[dst_v.at[j0 + 1]], add=True)
        return carry

    lax.fori_loop(0, NCHUNK // 2, body, 0)
    plsc.subcore_barrier()
    pltpu.sync_copy(acc_sh.at[pl.ds(sid * RPS, RPS)],
                    out_h.at[cid, pl.ds(sid * RPS, RPS)])


@functools.partial(
    pl.kernel,
    mesh=_mesh,
    compiler_params=pltpu.CompilerParams(needs_layout_passes=False, use_tc_tiling_on_sc=False),
    out_type=[
        jax.ShapeDtypeStruct((PR, 256), jnp.float32),
        jax.ShapeDtypeStruct((PR, 16), jnp.float32),
    ],
    scratch_types=[
        pltpu.VMEM((5, 125), jnp.int32),
        pltpu.VMEM((125, 256), jnp.float32),
        pltpu.VMEM((125, 16), jnp.float32),
        pltpu.VMEM_SHARED((PR, 256), jnp.float32),
        pltpu.VMEM_SHARED((PR, 16), jnp.float32),
    ],
)
def _sc_pool(y_h, cidx_h, zrow_h, one_h, z16_h, pool_h, cnt_h,
             cidx_v, rows_v, ones_v, acc_sh, cacc_sh):
    cid = lax.axis_index("c")
    sid = lax.axis_index("s")

    @pl.when(cid == 0)
    def _():
        pltpu.sync_copy(cidx_h.at[sid], cidx_v)
        pltpu.sync_copy(one_h, ones_v)
        pltpu.sync_copy(zrow_h, acc_sh.at[pl.ds(sid * 17, 17)])
        pltpu.sync_copy(z16_h, cacc_sh.at[pl.ds(sid * 17, 17)])

    plsc.subcore_barrier()

    @pl.when(cid == 0)
    def _():
        def body(j, carry):
            pltpu.sync_copy(y_h.at[pl.ds(sid * 625 + j * 125, 125)], rows_v)
            pltpu.sync_copy(rows_v, acc_sh.at[cidx_v.at[j]], add=True)
            pltpu.sync_copy(ones_v, cacc_sh.at[cidx_v.at[j]], add=True)
            return carry

        lax.fori_loop(0, 5, body, 0)

    plsc.subcore_barrier()

    @pl.when(cid == 0)
    def _():
        pltpu.sync_copy(acc_sh.at[pl.ds(sid * 17, 17)],
                        pool_h.at[pl.ds(sid * 17, 17)])
        pltpu.sync_copy(cacc_sh.at[pl.ds(sid * 17, 17)],
                        cnt_h.at[pl.ds(sid * 17, 17)])


def _dot_bf16(a, b):
    return jnp.dot(a.astype(jnp.bfloat16), b.astype(jnp.bfloat16),
                   preferred_element_type=jnp.float32)


def _elu(x):
    xn = jnp.minimum(x, 0.0)
    poly = xn * (1.0 + xn * (0.5 + xn * (1.0 / 6.0 + xn * (1.0 / 24.0
                 + xn * (1.0 / 120.0)))))
    em1 = jnp.where(xn > -0.125, poly, jnp.exp(xn) - 1.0)
    return jnp.where(x > 0, x, em1)


def _t1_body(x_ref, w_ref, degp_ref, xws_ref, o_dinv_ref):
    deg = jnp.sum(degp_ref[0], axis=0) + 1.0
    dv = 1.0 / jnp.sqrt(deg)
    xw = jnp.dot(x_ref[...].astype(jnp.bfloat16),
                 w_ref[...].astype(jnp.bfloat16),
                 preferred_element_type=jnp.float32)
    xws_ref[...] = dv[:, None] * xw
    o_dinv_ref[0] = jnp.broadcast_to(dv[None, :], (8, dv.shape[0]))


def _t2_body(acc_ref, xws_ref, b_ref, dinv_ref, hs_ref):
    dv = dinv_ref[0, 0, :]
    s = acc_ref[0] + acc_ref[1] + xws_ref[...]
    h = _elu(dv[:, None] * s + b_ref[...])
    hb = h.astype(jnp.bfloat16).astype(jnp.float32)
    hs_ref[...] = dv[:, None] * hb


def _t3_body(acc_ref, hs_ref, w_ref, b_ref, dinv_ref, h2_ref, s1_ref, s2_ref):
    i = pl.program_id(0)
    dv = dinv_ref[0, 0, :]
    ah = dv[:, None] * (acc_ref[0] + acc_ref[1] + hs_ref[...])
    wb = w_ref[...].astype(jnp.bfloat16).astype(jnp.float32)
    h2 = _elu(jnp.dot(ah, wb, precision=lax.Precision.HIGHEST,
                      preferred_element_type=jnp.float32)
              + b_ref[...])
    h2_ref[...] = h2

    @pl.when(i == 0)
    def _():
        s1_ref[...] = jnp.zeros_like(s1_ref)
        s2_ref[...] = jnp.zeros_like(s2_ref)

    s1_ref[...] += jnp.broadcast_to(jnp.sum(h2, axis=0)[None, :], s1_ref.shape)
    s2_ref[...] += jnp.broadcast_to(jnp.sum(h2 * h2, axis=0)[None, :],
                                    s2_ref.shape)


def _t4a_body(h2_ref, s1_ref, sq_ref):
    i = pl.program_id(0)
    mean = s1_ref[0, :] * (1.0 / N)
    d = h2_ref[...] - mean[None, :]

    @pl.when(i == 0)
    def _():
        sq_ref[...] = jnp.zeros_like(sq_ref)

    sq_ref[...] += jnp.broadcast_to(jnp.sum(d * d, axis=0)[None, :],
                                    sq_ref.shape)


def _t4_body(h2_ref, s1_ref, sq_ref, y_ref):
    mean = s1_ref[0, :] * (1.0 / N)
    var = sq_ref[0, :] * (1.0 / N)
    y_ref[...] = (h2_ref[...] - mean[None, :]) / jnp.sqrt(var + 1e-5)[None, :]


def _t5_body(bmp_ref, pool_ref, cnt_ref, wl1_ref, bl1_ref, wl2_ref, bl2_ref,
             wm1_ref, bm1_ref, wo1_ref, bo1_ref, g_ref):
    bm = jnp.sum(bmp_ref[...], axis=0)
    at = bm.T
    r = lax.broadcasted_iota(jnp.int32, (C, C), 0)
    c = lax.broadcasted_iota(jnp.int32, (C, C), 1)
    a = jnp.where(r == c, 1.0, jnp.where(at > 0, 1.0, 0.0))
    dc = 1.0 / jnp.sqrt(jnp.sum(a, axis=1))
    m = dc[:, None] * a * dc[None, :]
    px = pool_ref[0:C, :] / jnp.maximum(cnt_ref[0:C, 0:1], 1.0)
    z = _elu(jnp.dot(m, _dot_bf16(px, wl1_ref[...]),
                     precision=lax.Precision.HIGHEST, preferred_element_type=jnp.float32) + bl1_ref[...])
    z = _elu(jnp.dot(m, _dot_bf16(z, wl2_ref[...]),
                     precision=lax.Precision.HIGHEST, preferred_element_type=jnp.float32) + bl2_ref[...])
    z = _elu(jnp.dot(m, _dot_bf16(z, wm1_ref[...]),
                     precision=lax.Precision.HIGHEST, preferred_element_type=jnp.float32) + bm1_ref[...])
    g_ref[...] = jnp.dot(m, _dot_bf16(z, wo1_ref[...]),
                         precision=lax.Precision.HIGHEST, preferred_element_type=jnp.float32) + bo1_ref[...]


def _t6_body(w_ref, g_ref, o_ref):
    o_ref[...] = _dot_bf16(w_ref[...], g_ref[...])


def kernel(x, adj, in_batch, cluster, W_g1, b_g1, W_g2, b_g2, W_l1, b_l1,
           W_l2, b_l2, W_m1, b_m1, W_o1, b_o1, W_fc):
    f32 = jnp.float32
    src = adj[0]
    dst = adj[1]
    src2 = src.reshape(NW, EPW)
    dst2 = dst.reshape(NW, EPW)
    src3 = src.reshape(NW, NCHUNK, KE)
    dst3 = dst.reshape(NW, NCHUNK, KE)
    cidx3 = cluster.reshape(16, 5, 125)
    zbm = jnp.zeros((C * C,), f32)
    zrow = jnp.zeros((RPS, 64), f32)
    zrow256 = jnp.zeros((17, 256), f32)
    z16 = jnp.zeros((17, 16), f32)
    ones125 = jnp.ones((125, 16), f32)

    degT, bmp = _sc_precompute(src2, dst2, cluster, zbm)

    BN = 1000
    GN = N // BN
    t1 = pl.pallas_call(
        _t1_body,
        grid=(GN,),
        in_specs=[
            pl.BlockSpec((BN, 128), lambda i: (i, 0)),
            pl.BlockSpec((128, 64), lambda i: (0, 0)),
            pl.BlockSpec((1, NW, BN), lambda i: (i, 0, 0)),
        ],
        out_specs=[
            pl.BlockSpec((BN, 64), lambda i: (i, 0)),
            pl.BlockSpec((1, 8, BN), lambda i: (i, 0, 0)),
        ],
        out_shape=[
            jax.ShapeDtypeStruct((N, 64), f32),
            jax.ShapeDtypeStruct((GN, 8, BN), f32),
        ],
    )
    xws1, dinv8 = t1(x, W_g1, degT)

    accp1 = _sc_spmm(xws1, src3, dst3, zrow)

    t2 = pl.pallas_call(
        _t2_body,
        grid=(GN,),
        in_specs=[
            pl.BlockSpec((2, BN, 64), lambda i: (0, i, 0)),
            pl.BlockSpec((BN, 64), lambda i: (i, 0)),
            pl.BlockSpec((1, 64), lambda i: (0, 0)),
            pl.BlockSpec((1, 8, BN), lambda i: (i, 0, 0)),
        ],
        out_specs=pl.BlockSpec((BN, 64), lambda i: (i, 0)),
        out_shape=jax.ShapeDtypeStruct((N, 64), f32),
    )
    hs1 = t2(accp1, xws1, b_g1.reshape(1, 64), dinv8)

    accp2 = _sc_spmm(hs1, src3, dst3, zrow)

    t3 = pl.pallas_call(
        _t3_body,
        grid=(GN,),
        in_specs=[
            pl.BlockSpec((2, BN, 64), lambda i: (0, i, 0)),
            pl.BlockSpec((BN, 64), lambda i: (i, 0)),
            pl.BlockSpec((64, 256), lambda i: (0, 0)),
            pl.BlockSpec((1, 256), lambda i: (0, 0)),
            pl.BlockSpec((1, 8, BN), lambda i: (i, 0, 0)),
        ],
        out_specs=[
            pl.BlockSpec((BN, 256), lambda i: (i, 0)),
            pl.BlockSpec((8, 256), lambda i: (0, 0)),
            pl.BlockSpec((8, 256), lambda i: (0, 0)),
        ],
        out_shape=[
            jax.ShapeDtypeStruct((N, 256), f32),
            jax.ShapeDtypeStruct((8, 256), f32),
            jax.ShapeDtypeStruct((8, 256), f32),
        ],
    )
    h2, s1, s2 = t3(accp2, hs1, W_g2, b_g2.reshape(1, 256), dinv8)

    t4a = pl.pallas_call(
        _t4a_body,
        grid=(GN,),
        in_specs=[
            pl.BlockSpec((BN, 256), lambda i: (i, 0)),
            pl.BlockSpec((8, 256), lambda i: (0, 0)),
        ],
        out_specs=pl.BlockSpec((8, 256), lambda i: (0, 0)),
        out_shape=jax.ShapeDtypeStruct((8, 256), f32),
    )
    sq = t4a(h2, s1)

    t4 = pl.pallas_call(
        _t4_body,
        grid=(GN,),
        in_specs=[
            pl.BlockSpec((BN, 256), lambda i: (i, 0)),
            pl.BlockSpec((8, 256), lambda i: (0, 0)),
            pl.BlockSpec((8, 256), lambda i: (0, 0)),
        ],
        out_specs=pl.BlockSpec((BN, 256), lambda i: (i, 0)),
        out_shape=jax.ShapeDtypeStruct((N, 256), f32),
    )
    y = t4(h2, s1, sq)

    pool, cntl = _sc_pool(y, cidx3, zrow256, ones125, z16)

    t5 = pl.pallas_call(
        _t5_body,
        in_specs=[
            pl.BlockSpec((NW, C, C), lambda: (0, 0, 0)),
            pl.BlockSpec((PR, 256), lambda: (0, 0)),
            pl.BlockSpec((PR, 16), lambda: (0, 0)),
            pl.BlockSpec((256, 128), lambda: (0, 0)),
            pl.BlockSpec((1, 128), lambda: (0, 0)),
            pl.BlockSpec((128, 64), lambda: (0, 0)),
            pl.BlockSpec((1, 64), lambda: (0, 0)),
            pl.BlockSpec((64, 16), lambda: (0, 0)),
            pl.BlockSpec((1, 16), lambda: (0, 0)),
            pl.BlockSpec((16, 8), lambda: (0, 0)),
            pl.BlockSpec((1, 8), lambda: (0, 0)),
        ],
        out_specs=pl.BlockSpec((C, 8), lambda: (0, 0)),
        out_shape=jax.ShapeDtypeStruct((C, 8), f32),
    )
    wo1p = jnp.pad(W_o1, ((0, 0), (0, 3)))
    bo1p = jnp.pad(b_o1, (0, 3)).reshape(1, 8)
    gpad = t5(bmp.reshape(NW, C, C), pool, cntl, W_l1, b_l1.reshape(1, 128),
              W_l2, b_l2.reshape(1, 64), W_m1, b_m1.reshape(1, 16),
              wo1p, bo1p)
    g = gpad[:, :5].reshape(-1)

    BM = 1000
    t6 = pl.pallas_call(
        _t6_body,
        grid=(30000 // BM,),
        in_specs=[
            pl.BlockSpec((BM, 1280), lambda i: (i, 0)),
            pl.BlockSpec((1280, 1), lambda i: (0, 0)),
        ],
        out_specs=pl.BlockSpec((BM, 1), lambda i: (i, 0)),
        out_shape=jax.ShapeDtypeStruct((30000, 1), f32),
    )
    o = t6(W_fc, g.reshape(1280, 1)).reshape(-1)
    return (g, o)
